# baseline (speedup 1.0000x reference)
import jax
import jax.numpy as jnp
from jax import lax
from jax.experimental import pallas as pl
from jax.experimental.pallas import tpu as pltpu

B, N = 16, 4096
X_DIM, Y_DIM, H_DIM, R_DIM = 16, 16, 64, 64
PACK = 128 // X_DIM
PROWS = N // PACK
NPAIR = PACK // 2
K = 4
CROWS = PROWS // K
T = B * K
NBUF = 6


def _bd_mask(rows, cols, rblk, cblk):
    ri = lax.broadcasted_iota(jnp.int32, (rows, cols), 0) // rblk
    ci = lax.broadcasted_iota(jnp.int32, (rows, cols), 1) // cblk
    return (ri == ci).astype(jnp.float32)


def _copies(t, slot, xh, yh, mh, xb, yb, mb, sems):
    b, k = divmod(t, K)
    row0 = k * CROWS
    return (
        pltpu.make_async_copy(
            xh.at[b, pl.ds(row0, CROWS), :], xb.at[slot], sems.at[slot, 0]),
        pltpu.make_async_copy(
            yh.at[b, pl.ds(row0, CROWS), :], yb.at[slot], sems.at[slot, 1]),
        pltpu.make_async_copy(
            mh.at[b, pl.ds(row0, CROWS), :], mb.at[slot], sems.at[slot, 2]),
    )


def _body(xh, yh, mh, w1_ref, b1_ref, w2_ref, b2_ref, w3_ref, b3_ref,
          out_ref, xb, yb, mb, sems):
    w1 = w1_ref[...]
    w1x_bd = jnp.tile(w1[:X_DIM], (PACK, PACK)) * _bd_mask(128, 512, 16, 64)
    w1y_bd = jnp.tile(w1[X_DIM:], (PACK, PACK)) * _bd_mask(128, 512, 16, 64)
    w2_bd = jnp.tile(w2_ref[...], (2, 2)) * _bd_mask(128, 128, 64, 64)
    b1t = jnp.tile(b1_ref[...], (1, PACK))
    b2t = jnp.tile(b2_ref[...], (1, 2))
    e_mat = _bd_mask(PACK, PACK * H_DIM, 1, H_DIM)

    for t in range(min(NBUF, T)):
        for i, c in enumerate(_copies(t, t % NBUF, xh, yh, mh, xb, yb, mb,
                                      sems)):
            c.start(priority=t % 2)

    acc = jnp.zeros((1, 2 * H_DIM), dtype=jnp.float32)
    cnt = jnp.zeros((), dtype=jnp.float32)
    for t in range(T):
        slot = t % NBUF
        b, k = divmod(t, K)
        for c in _copies(t, slot, xh, yh, mh, xb, yb, mb, sems):
            c.wait()
        px = xb[slot]
        py = yb[slot]
        mp = mb[slot]

        h = jnp.dot(px, w1x_bd, preferred_element_type=jnp.float32)
        h = h + jnp.dot(py, w1y_bd, preferred_element_type=jnp.float32)
        h = jnp.maximum(h + b1t, 0.0)
        mexp = jnp.dot(mp, e_mat, preferred_element_type=jnp.float32)
        part = jnp.zeros((1, 2 * H_DIM), dtype=jnp.float32)
        for p in range(NPAIR):
            g = h[:, 2 * H_DIM * p:2 * H_DIM * (p + 1)]
            h2 = jnp.dot(g, w2_bd, preferred_element_type=jnp.float32)
            h2 = jnp.maximum(h2 + b2t, 0.0)
            mm = mexp[:, 2 * H_DIM * p:2 * H_DIM * (p + 1)]
            part = part + jnp.sum(h2 * mm, axis=0, keepdims=True)
        acc = acc + part
        cnt = cnt + jnp.sum(mp)

        if k == K - 1:
            s = acc[:, :H_DIM] + acc[:, H_DIM:]
            r = jnp.dot(s, w3_ref[...], preferred_element_type=jnp.float32)
            r = r + cnt * b3_ref[...]
            out_ref[pl.ds(b, 1), :] = r / jnp.maximum(cnt, 1.0)
            acc = jnp.zeros((1, 2 * H_DIM), dtype=jnp.float32)
            cnt = jnp.zeros((), dtype=jnp.float32)

        nxt = t + NBUF
        if nxt < T:
            for i, c in enumerate(_copies(nxt, slot, xh, yh, mh, xb, yb, mb,
                                          sems)):
                c.start(priority=nxt % 2)


def kernel(x, y, mask, W1, b1, W2, b2, W3, b3):
    xd = x.reshape(B, PROWS, 128)
    yd = y.reshape(B, PROWS, 128)
    mp = mask.astype(jnp.float32).reshape(B, PROWS, PACK)
    b1r = b1.reshape(1, H_DIM)
    b2r = b2.reshape(1, H_DIM)
    b3r = b3.reshape(1, R_DIM)

    out = pl.pallas_call(
        _body,
        in_specs=[
            pl.BlockSpec(memory_space=pl.ANY),
            pl.BlockSpec(memory_space=pl.ANY),
            pl.BlockSpec(memory_space=pl.ANY),
            pl.BlockSpec((X_DIM + Y_DIM, H_DIM), lambda: (0, 0)),
            pl.BlockSpec((1, H_DIM), lambda: (0, 0)),
            pl.BlockSpec((H_DIM, H_DIM), lambda: (0, 0)),
            pl.BlockSpec((1, H_DIM), lambda: (0, 0)),
            pl.BlockSpec((H_DIM, R_DIM), lambda: (0, 0)),
            pl.BlockSpec((1, R_DIM), lambda: (0, 0)),
        ],
        out_specs=pl.BlockSpec((B, R_DIM), lambda: (0, 0)),
        out_shape=jax.ShapeDtypeStruct((B, R_DIM), jnp.float32),
        scratch_shapes=[
            pltpu.VMEM((NBUF, CROWS, 128), jnp.float32),
            pltpu.VMEM((NBUF, CROWS, 128), jnp.float32),
            pltpu.VMEM((NBUF, CROWS, PACK), jnp.float32),
            pltpu.SemaphoreType.DMA((NBUF, 3)),
        ],
    )(xd, yd, mp, W1, b1r, W2, b2r, W3, b3r)
    return out


# speedup vs baseline: 1.1456x; 1.1456x over previous
import jax
import jax.numpy as jnp
from jax import lax
from jax.experimental import pallas as pl
from jax.experimental.pallas import tpu as pltpu

B, N = 16, 4096
X_DIM, Y_DIM, H_DIM, R_DIM = 16, 16, 64, 64
PACK = 128 // X_DIM
PROWS = N // PACK
NPAIR = PACK // 2
GB = 4
NCH = B // GB
CHROWS = GB * PROWS


def _bd_mask(rows, cols, rblk, cblk):
    ri = lax.broadcasted_iota(jnp.int32, (rows, cols), 0) // rblk
    ci = lax.broadcasted_iota(jnp.int32, (rows, cols), 1) // cblk
    return (ri == ci).astype(jnp.float32)


def _chunk_copies(c, xh, yh, mh, xv, yv, mv, sems):
    rs = pl.ds(c * CHROWS, CHROWS)
    bs = pl.ds(c * GB, GB)
    return (
        pltpu.make_async_copy(xh.at[rs, :], xv.at[rs, :], sems.at[c, 0]),
        pltpu.make_async_copy(yh.at[rs, :], yv.at[rs, :], sems.at[c, 1]),
        pltpu.make_async_copy(mh.at[bs], mv.at[bs], sems.at[c, 2]),
    )


def _body(xh, yh, mh, w1_ref, b1_ref, w2_ref, b2_ref, w3_ref, b3_ref,
          out_ref, xv, yv, mv, sems):
    for c in range(NCH):
        for cp in _chunk_copies(c, xh, yh, mh, xv, yv, mv, sems):
            cp.start()

    w1 = w1_ref[...]
    w1x_bd = jnp.tile(w1[:X_DIM], (PACK, PACK)) * _bd_mask(128, 512, 16, 64)
    w1y_bd = jnp.tile(w1[X_DIM:], (PACK, PACK)) * _bd_mask(128, 512, 16, 64)
    w2_bd = jnp.tile(w2_ref[...], (2, 2)) * _bd_mask(128, 128, 64, 64)
    b1t = jnp.tile(b1_ref[...], (1, PACK))
    b2t = jnp.tile(b2_ref[...], (1, 2))
    e_mat = _bd_mask(PACK, PACK * H_DIM, 1, H_DIM)

    for c in range(NCH):
        for cp in _chunk_copies(c, xh, yh, mh, xv, yv, mv, sems):
            cp.wait()
        for bi in range(GB):
            b = c * GB + bi
            rs = pl.ds(b * PROWS, PROWS)
            px = xv[rs, :]
            py = yv[rs, :]
            mp = mv[b]

            h = jnp.dot(px, w1x_bd, preferred_element_type=jnp.float32)
            h = h + jnp.dot(py, w1y_bd, preferred_element_type=jnp.float32)
            h = jnp.maximum(h + b1t, 0.0)
            mexp = jnp.dot(mp, e_mat, preferred_element_type=jnp.float32)
            acc = jnp.zeros((1, 2 * H_DIM), dtype=jnp.float32)
            for p in range(NPAIR):
                g = h[:, 2 * H_DIM * p:2 * H_DIM * (p + 1)]
                h2 = jnp.dot(g, w2_bd, preferred_element_type=jnp.float32)
                h2 = jnp.maximum(h2 + b2t, 0.0)
                mm = mexp[:, 2 * H_DIM * p:2 * H_DIM * (p + 1)]
                acc = acc + jnp.sum(h2 * mm, axis=0, keepdims=True)
            s = acc[:, :H_DIM] + acc[:, H_DIM:]
            cnt = jnp.sum(mp)
            r = jnp.dot(s, w3_ref[...], preferred_element_type=jnp.float32)
            r = r + cnt * b3_ref[...]
            out_ref[pl.ds(b, 1), :] = r / jnp.maximum(cnt, 1.0)


def kernel(x, y, mask, W1, b1, W2, b2, W3, b3):
    xd = x.reshape(B * PROWS, 128)
    yd = y.reshape(B * PROWS, 128)
    mp = mask.astype(jnp.float32).reshape(B, PROWS, PACK)
    b1r = b1.reshape(1, H_DIM)
    b2r = b2.reshape(1, H_DIM)
    b3r = b3.reshape(1, R_DIM)

    out = pl.pallas_call(
        _body,
        in_specs=[
            pl.BlockSpec(memory_space=pl.ANY),
            pl.BlockSpec(memory_space=pl.ANY),
            pl.BlockSpec(memory_space=pl.ANY),
            pl.BlockSpec((X_DIM + Y_DIM, H_DIM), lambda: (0, 0)),
            pl.BlockSpec((1, H_DIM), lambda: (0, 0)),
            pl.BlockSpec((H_DIM, H_DIM), lambda: (0, 0)),
            pl.BlockSpec((1, H_DIM), lambda: (0, 0)),
            pl.BlockSpec((H_DIM, R_DIM), lambda: (0, 0)),
            pl.BlockSpec((1, R_DIM), lambda: (0, 0)),
        ],
        out_specs=pl.BlockSpec((B, R_DIM), lambda: (0, 0)),
        out_shape=jax.ShapeDtypeStruct((B, R_DIM), jnp.float32),
        scratch_shapes=[
            pltpu.VMEM((B * PROWS, 128), jnp.float32),
            pltpu.VMEM((B * PROWS, 128), jnp.float32),
            pltpu.VMEM((B, PROWS, PACK), jnp.float32),
            pltpu.SemaphoreType.DMA((NCH, 3)),
        ],
    )(xd, yd, mp, W1, b1r, W2, b2r, W3, b3r)
    return out
